# baseline (speedup 1.0000x reference)
import dataclasses
import functools

import jax
import jax.numpy as jnp
from jax import lax
from jax.experimental import pallas as pl
from jax.experimental.pallas import tpu as pltpu
from jax.experimental.pallas import tpu_sc as plsc

N = 10000
NP = 10240
E = 320000
D = 128
H = 64
C = 10
CP = 16

NC = 2
NS = 16
NW = NC * NS
EPW = E // NW
B = 80
NCH = EPW // B
NPS = NP // NS

_mesh = plsc.VectorSubcoreMesh(core_axis_name="c", subcore_axis_name="s")
_f32 = jnp.float32

_sc_params = pltpu.CompilerParams(
    needs_layout_passes=False, use_tc_tiling_on_sc=False)


def _deg_body(dst_hbm, w_hbm, out_hbm, dst_v, w_v, z_v, acc_sh):
    c = lax.axis_index("c")
    s = lax.axis_index("s")
    wid = s * NC + c

    @pl.loop(0, NPS, step=16)
    def _(i):
        z_v[pl.ds(i, 16)] = jnp.zeros((16,), _f32)

    pltpu.sync_copy(z_v, acc_sh.at[pl.ds(s * NPS, NPS)])
    plsc.subcore_barrier()

    pltpu.sync_copy(dst_hbm.at[wid], dst_v)
    pltpu.sync_copy(w_hbm.at[wid], w_v)

    @pl.loop(0, NCH)
    def _(ci):
        pltpu.sync_copy(w_v.at[ci], acc_sh.at[dst_v.at[ci]], add=True)

    plsc.subcore_barrier()
    pltpu.sync_copy(acc_sh.at[pl.ds(s * NPS, NPS)],
                    out_hbm.at[c, pl.ds(s * NPS, NPS)])


@functools.partial(
    pl.kernel,
    out_type=jax.ShapeDtypeStruct((NC, NP), _f32),
    mesh=_mesh,
    scratch_types=[
        pltpu.VMEM((NCH, B), jnp.int32),
        pltpu.VMEM((NCH, B), _f32),
        pltpu.VMEM((NPS,), _f32),
        pltpu.VMEM_SHARED((NP,), _f32),
    ],
    compiler_params=_sc_params,
)
def _deg_kernel(dst_hbm, w_hbm, out_hbm, dst_v, w_v, z_v, acc_sh):
    _deg_body(dst_hbm, w_hbm, out_hbm, dst_v, w_v, z_v, acc_sh)


NBUF = 5


def _msg_body(wd, hs_hbm, src_hbm, dst_hbm, w_hbm, out_hbm,
              src_v, dst_v, w_v, srows, rows, z_v, acc_sh, gsem, ssem):
    c = lax.axis_index("c")
    s = lax.axis_index("s")
    wid = s * NC + c

    @pl.loop(0, B)
    def _(r):
        for q in range(wd // 16):
            z_v[r, pl.ds(q * 16, 16)] = jnp.zeros((16,), _f32)

    @pl.loop(0, NPS // B)
    def _(j):
        pltpu.sync_copy(z_v, acc_sh.at[pl.ds(s * NPS + j * B, B)])

    plsc.subcore_barrier()

    pltpu.async_copy(src_hbm.at[wid], src_v, gsem.at[0])
    pltpu.async_copy(dst_hbm.at[wid], dst_v, gsem.at[1])
    pltpu.async_copy(w_hbm.at[wid], w_v, gsem.at[2])
    pltpu.make_async_copy(src_hbm.at[0], src_v, gsem.at[0]).wait()
    pltpu.make_async_copy(dst_hbm.at[0], dst_v, gsem.at[1]).wait()
    pltpu.make_async_copy(w_hbm.at[0], w_v, gsem.at[2]).wait()

    def gstart(ci, b):
        pltpu.async_copy(hs_hbm.at[src_v.at[ci]], srows.at[b], gsem.at[b])

    def gwait(b):
        pltpu.make_async_copy(hs_hbm.at[src_v.at[0]], srows.at[b],
                              gsem.at[b]).wait()

    def sstart(ci, b):
        pltpu.async_copy(rows.at[b], acc_sh.at[dst_v.at[ci]], ssem.at[b],
                         add=True)

    def swait(b):
        pltpu.make_async_copy(rows.at[b], acc_sh.at[dst_v.at[0]],
                              ssem.at[b]).wait()

    def scale(ci, b):
        ci_idx = jnp.full((16,), ci, jnp.int32)
        for r in range(B):
            splat = plsc.load_gather(
                w_v, [ci_idx, jnp.full((16,), r, jnp.int32)])
            for q in range(wd // 16):
                rows[b, r, pl.ds(q * 16, 16)] = (
                    srows[b, r, pl.ds(q * 16, 16)] * splat)

    for b in range(NBUF):
        gstart(b, b)

    @pl.loop(0, NCH - NBUF, step=NBUF)
    def _(c0):
        for b in range(NBUF):
            gwait(b)
            scale(c0 + b, b)
            sstart(c0 + b, b)
        for b in range(NBUF):
            swait(b)
            gstart(c0 + NBUF + b, b)

    for b in range(NBUF):
        gwait(b)
        scale(NCH - NBUF + b, b)
        sstart(NCH - NBUF + b, b)
    for b in range(NBUF):
        swait(b)

    plsc.subcore_barrier()
    pltpu.sync_copy(acc_sh.at[pl.ds(s * NPS, NPS)],
                    out_hbm.at[c, pl.ds(s * NPS, NPS)])


def _make_msg_kernel(wd):
    @functools.partial(
        pl.kernel,
        out_type=jax.ShapeDtypeStruct((NC, NP, wd), _f32),
        mesh=_mesh,
        scratch_types=[
            pltpu.VMEM((NCH, B), jnp.int32),
            pltpu.VMEM((NCH, B), jnp.int32),
            pltpu.VMEM((NCH, B), _f32),
            pltpu.VMEM((NBUF, B, wd), _f32),
            pltpu.VMEM((NBUF, B, wd), _f32),
            pltpu.VMEM((B, wd), _f32),
            pltpu.VMEM_SHARED((NP, wd), _f32),
            pltpu.SemaphoreType.DMA((NBUF,)),
            pltpu.SemaphoreType.DMA((NBUF,)),
        ],
        compiler_params=_sc_params,
    )
    def _k(hs_hbm, src_hbm, dst_hbm, w_hbm, out_hbm,
           src_v, dst_v, w_v, srows, rows, z_v, acc_sh, gsem, ssem):
        _msg_body(wd, hs_hbm, src_hbm, dst_hbm, w_hbm, out_hbm,
                  src_v, dst_v, w_v, srows, rows, z_v, acc_sh, gsem, ssem)
    return _k


_msg_kernel_h = _make_msg_kernel(H)
_msg_kernel_c = _make_msg_kernel(CP)


def _tc1_body(x_ref, w1_ref, degp_ref, hs_ref):
    deg = degp_ref[0] + degp_ref[1] + 1.0
    dinv = lax.rsqrt(deg)
    h = jnp.dot(x_ref[...], w1_ref[...], preferred_element_type=_f32)
    hs_ref[...] = h * dinv


def _tc2_body(acc_ref, hs_ref, degp_ref, w2_ref, b1_ref, hs2_ref):
    deg = degp_ref[0] + degp_ref[1] + 1.0
    dinv = lax.rsqrt(deg)
    t = jax.nn.relu(dinv * (acc_ref[0] + acc_ref[1] + hs_ref[...])
                    + b1_ref[...])
    hs2_ref[...] = jnp.dot(t, w2_ref[...], preferred_element_type=_f32) * dinv


def _tc3_body(acc_ref, hs2_ref, degp_ref, b2_ref, lp_ref, xo_ref):
    deg = degp_ref[0] + degp_ref[1] + 1.0
    dinv = lax.rsqrt(deg)
    xo = dinv * (acc_ref[0] + acc_ref[1] + hs2_ref[...]) + b2_ref[...]
    col = lax.broadcasted_iota(jnp.int32, (NP, CP), 1)
    masked = jnp.where(col < C, xo, -1e30)
    m = jnp.max(masked, axis=1, keepdims=True)
    ssum = jnp.sum(jnp.exp(masked - m), axis=1, keepdims=True)
    lp_ref[...] = xo - m - jnp.log(ssum)
    xo_ref[...] = xo


def kernel(x, edge_index, e_w, idx, W1, b1, W2, b2):
    w = jnp.where(idx == 0, jnp.ones((E,), x.dtype), e_w)
    src3 = edge_index[0].reshape(NW, NCH, B)
    dst3 = edge_index[1].reshape(NW, NCH, B)
    w3 = w.reshape(NW, NCH, B)

    x_pad = jnp.pad(x, ((0, NP - N), (0, 0)))
    w2p = jnp.pad(W2, ((0, 0), (0, CP - C)))
    b1r = b1.reshape(1, H)
    b2r = jnp.pad(b2, (0, CP - C)).reshape(1, CP)

    degp = _deg_kernel(dst3, w3)
    degp3 = degp.reshape(NC, NP, 1)

    hs = pl.pallas_call(
        _tc1_body,
        out_shape=jax.ShapeDtypeStruct((NP, H), _f32),
    )(x_pad, W1, degp3)

    acc1 = _msg_kernel_h(hs, src3, dst3, w3)

    hs2 = pl.pallas_call(
        _tc2_body,
        out_shape=jax.ShapeDtypeStruct((NP, CP), _f32),
    )(acc1, hs, degp3, w2p, b1r)

    acc2 = _msg_kernel_c(hs2, src3, dst3, w3)

    lp, xo = pl.pallas_call(
        _tc3_body,
        out_shape=[jax.ShapeDtypeStruct((NP, CP), _f32),
                   jax.ShapeDtypeStruct((NP, CP), _f32)],
    )(acc2, hs2, degp3, b2r)

    log_probs = lp[:N, :C]
    x_out = xo[:N, :C]
    preg = jnp.asarray(0.0, dtype=_f32)
    return (log_probs, x_out, preg)


# speedup vs baseline: 1.5892x; 1.5892x over previous
import dataclasses
import functools

import jax
import jax.numpy as jnp
from jax import lax
from jax.experimental import pallas as pl
from jax.experimental.pallas import tpu as pltpu
from jax.experimental.pallas import tpu_sc as plsc

N = 10000
NP = 10240
E = 320000
D = 128
H = 64
C = 10
CP = 16

NC = 2
NS = 16
NW = NC * NS
EPW = E // NW
B = 80
NCH = EPW // B
NPS = NP // NS

_mesh = plsc.VectorSubcoreMesh(core_axis_name="c", subcore_axis_name="s")
_f32 = jnp.float32

_sc_params = pltpu.CompilerParams(
    needs_layout_passes=False, use_tc_tiling_on_sc=False)


def _lane_splat(vec16, j):
    return lax.gather(
        vec16,
        jnp.full((16, 1), j, jnp.int32),
        lax.GatherDimensionNumbers(
            offset_dims=(), collapsed_slice_dims=(0,), start_index_map=(0,)),
        (1,),
        mode=lax.GatherScatterMode.PROMISE_IN_BOUNDS,
    )


def _deg_body(dst_hbm, w_hbm, out_hbm, dst_v, w_v, z_v, acc_sh):
    c = lax.axis_index("c")
    s = lax.axis_index("s")
    wid = s * NC + c

    @pl.loop(0, NPS, step=16)
    def _(i):
        z_v[pl.ds(i, 16)] = jnp.zeros((16,), _f32)

    pltpu.sync_copy(z_v, acc_sh.at[pl.ds(s * NPS, NPS)])
    plsc.subcore_barrier()

    pltpu.sync_copy(dst_hbm.at[wid], dst_v)
    pltpu.sync_copy(w_hbm.at[wid], w_v)

    @pl.loop(0, NCH)
    def _(ci):
        pltpu.sync_copy(w_v.at[ci], acc_sh.at[dst_v.at[ci]], add=True)

    plsc.subcore_barrier()
    pltpu.sync_copy(acc_sh.at[pl.ds(s * NPS, NPS)],
                    out_hbm.at[c, pl.ds(s * NPS, NPS)])


@functools.partial(
    pl.kernel,
    out_type=jax.ShapeDtypeStruct((NC, NP), _f32),
    mesh=_mesh,
    scratch_types=[
        pltpu.VMEM((NCH, B), jnp.int32),
        pltpu.VMEM((NCH, B), _f32),
        pltpu.VMEM((NPS,), _f32),
        pltpu.VMEM_SHARED((NP,), _f32),
    ],
    compiler_params=_sc_params,
)
def _deg_kernel(dst_hbm, w_hbm, out_hbm, dst_v, w_v, z_v, acc_sh):
    _deg_body(dst_hbm, w_hbm, out_hbm, dst_v, w_v, z_v, acc_sh)


NBUF = 5


def _msg_body(wd, hs_hbm, src_hbm, dst_hbm, w_hbm, out_hbm,
              src_v, dst_v, w_v, srows, rows, z_v, acc_sh, gsem, ssem):
    c = lax.axis_index("c")
    s = lax.axis_index("s")
    wid = s * NC + c

    @pl.loop(0, B)
    def _(r):
        for q in range(wd // 16):
            z_v[r, pl.ds(q * 16, 16)] = jnp.zeros((16,), _f32)

    @pl.loop(0, NPS // B)
    def _(j):
        pltpu.sync_copy(z_v, acc_sh.at[pl.ds(s * NPS + j * B, B)])

    plsc.subcore_barrier()

    pltpu.async_copy(src_hbm.at[wid], src_v, gsem.at[0])
    pltpu.async_copy(dst_hbm.at[wid], dst_v, gsem.at[1])
    pltpu.async_copy(w_hbm.at[wid], w_v, gsem.at[2])
    pltpu.make_async_copy(src_hbm.at[0], src_v, gsem.at[0]).wait()
    pltpu.make_async_copy(dst_hbm.at[0], dst_v, gsem.at[1]).wait()
    pltpu.make_async_copy(w_hbm.at[0], w_v, gsem.at[2]).wait()

    def gstart(ci, b):
        pltpu.async_copy(hs_hbm.at[src_v.at[ci]], srows.at[b], gsem.at[b])

    def gwait(b):
        pltpu.make_async_copy(hs_hbm.at[src_v.at[0]], srows.at[b],
                              gsem.at[b]).wait()

    def sstart(ci, b):
        pltpu.async_copy(rows.at[b], acc_sh.at[dst_v.at[ci]], ssem.at[b],
                         add=True)

    def swait(b):
        pltpu.make_async_copy(rows.at[b], acc_sh.at[dst_v.at[0]],
                              ssem.at[b]).wait()

    def scale(ci, b):
        for g in range(B // 16):
            wv = w_v[ci, pl.ds(g * 16, 16)]
            for j in range(16):
                splat = _lane_splat(wv, j)
                r = g * 16 + j
                for q in range(wd // 16):
                    rows[b, r, pl.ds(q * 16, 16)] = (
                        srows[b, r, pl.ds(q * 16, 16)] * splat)

    for b in range(NBUF):
        gstart(b, b)

    @pl.loop(0, NCH - NBUF, step=NBUF)
    def _(c0):
        for b in range(NBUF):
            gwait(b)
            scale(c0 + b, b)
            sstart(c0 + b, b)
        for b in range(NBUF):
            swait(b)
            gstart(c0 + NBUF + b, b)

    for b in range(NBUF):
        gwait(b)
        scale(NCH - NBUF + b, b)
        sstart(NCH - NBUF + b, b)
    for b in range(NBUF):
        swait(b)

    plsc.subcore_barrier()
    pltpu.sync_copy(acc_sh.at[pl.ds(s * NPS, NPS)],
                    out_hbm.at[c, pl.ds(s * NPS, NPS)])


def _make_msg_kernel(wd):
    @functools.partial(
        pl.kernel,
        out_type=jax.ShapeDtypeStruct((NC, NP, wd), _f32),
        mesh=_mesh,
        scratch_types=[
            pltpu.VMEM((NCH, B), jnp.int32),
            pltpu.VMEM((NCH, B), jnp.int32),
            pltpu.VMEM((NCH, B), _f32),
            pltpu.VMEM((NBUF, B, wd), _f32),
            pltpu.VMEM((NBUF, B, wd), _f32),
            pltpu.VMEM((B, wd), _f32),
            pltpu.VMEM_SHARED((NP, wd), _f32),
            pltpu.SemaphoreType.DMA((NBUF,)),
            pltpu.SemaphoreType.DMA((NBUF,)),
        ],
        compiler_params=_sc_params,
    )
    def _k(hs_hbm, src_hbm, dst_hbm, w_hbm, out_hbm,
           src_v, dst_v, w_v, srows, rows, z_v, acc_sh, gsem, ssem):
        _msg_body(wd, hs_hbm, src_hbm, dst_hbm, w_hbm, out_hbm,
                  src_v, dst_v, w_v, srows, rows, z_v, acc_sh, gsem, ssem)
    return _k


_msg_kernel_h = _make_msg_kernel(H)
_msg_kernel_c = _make_msg_kernel(CP)


def _tc1_body(x_ref, w1_ref, degp_ref, hs_ref):
    deg = degp_ref[0] + degp_ref[1] + 1.0
    dinv = lax.rsqrt(deg)
    h = jnp.dot(x_ref[...], w1_ref[...], preferred_element_type=_f32)
    hs_ref[...] = h * dinv


def _tc2_body(acc_ref, hs_ref, degp_ref, w2_ref, b1_ref, hs2_ref):
    deg = degp_ref[0] + degp_ref[1] + 1.0
    dinv = lax.rsqrt(deg)
    t = jax.nn.relu(dinv * (acc_ref[0] + acc_ref[1] + hs_ref[...])
                    + b1_ref[...])
    hs2_ref[...] = jnp.dot(t, w2_ref[...], preferred_element_type=_f32) * dinv


def _tc3_body(acc_ref, hs2_ref, degp_ref, b2_ref, lp_ref, xo_ref):
    deg = degp_ref[0] + degp_ref[1] + 1.0
    dinv = lax.rsqrt(deg)
    xo = dinv * (acc_ref[0] + acc_ref[1] + hs2_ref[...]) + b2_ref[...]
    col = lax.broadcasted_iota(jnp.int32, (NP, CP), 1)
    masked = jnp.where(col < C, xo, -1e30)
    m = jnp.max(masked, axis=1, keepdims=True)
    ssum = jnp.sum(jnp.exp(masked - m), axis=1, keepdims=True)
    lp_ref[...] = xo - m - jnp.log(ssum)
    xo_ref[...] = xo


def kernel(x, edge_index, e_w, idx, W1, b1, W2, b2):
    w = jnp.where(idx == 0, jnp.ones((E,), x.dtype), e_w)
    src3 = edge_index[0].reshape(NW, NCH, B)
    dst3 = edge_index[1].reshape(NW, NCH, B)
    w3 = w.reshape(NW, NCH, B)

    x_pad = jnp.pad(x, ((0, NP - N), (0, 0)))
    w2p = jnp.pad(W2, ((0, 0), (0, CP - C)))
    b1r = b1.reshape(1, H)
    b2r = jnp.pad(b2, (0, CP - C)).reshape(1, CP)

    degp = _deg_kernel(dst3, w3)
    degp3 = degp.reshape(NC, NP, 1)

    hs = pl.pallas_call(
        _tc1_body,
        out_shape=jax.ShapeDtypeStruct((NP, H), _f32),
    )(x_pad, W1, degp3)

    acc1 = _msg_kernel_h(hs, src3, dst3, w3)

    hs2 = pl.pallas_call(
        _tc2_body,
        out_shape=jax.ShapeDtypeStruct((NP, CP), _f32),
    )(acc1, hs, degp3, w2p, b1r)

    acc2 = _msg_kernel_c(hs2, src3, dst3, w3)

    lp, xo = pl.pallas_call(
        _tc3_body,
        out_shape=[jax.ShapeDtypeStruct((NP, CP), _f32),
                   jax.ShapeDtypeStruct((NP, CP), _f32)],
    )(acc2, hs2, degp3, b2r)

    log_probs = lp[:N, :C]
    x_out = xo[:N, :C]
    preg = jnp.asarray(0.0, dtype=_f32)
    return (log_probs, x_out, preg)


# speedup vs baseline: 1.6043x; 1.0095x over previous
import dataclasses
import functools

import jax
import jax.numpy as jnp
from jax import lax
from jax.experimental import pallas as pl
from jax.experimental.pallas import tpu as pltpu
from jax.experimental.pallas import tpu_sc as plsc

N = 10000
NP = 10240
E = 320000
D = 128
H = 64
C = 10
CP = 16

NC = 2
NS = 16
NW = NC * NS
EPW = E // NW
B = 80
NCH = EPW // B
NPS = NP // NS

_mesh = plsc.VectorSubcoreMesh(core_axis_name="c", subcore_axis_name="s")
_f32 = jnp.float32

_sc_params = pltpu.CompilerParams(
    needs_layout_passes=False, use_tc_tiling_on_sc=False)


def _lane_splat(vec16, j):
    return lax.gather(
        vec16,
        jnp.full((16, 1), j, jnp.int32),
        lax.GatherDimensionNumbers(
            offset_dims=(), collapsed_slice_dims=(0,), start_index_map=(0,)),
        (1,),
        mode=lax.GatherScatterMode.PROMISE_IN_BOUNDS,
    )


def _deg_body(dst_hbm, w_hbm, out_hbm, dst_v, w_v, z_v, acc_sh):
    c = lax.axis_index("c")
    s = lax.axis_index("s")
    wid = s * NC + c

    @pl.loop(0, NPS, step=16)
    def _(i):
        z_v[pl.ds(i, 16)] = jnp.zeros((16,), _f32)

    pltpu.sync_copy(z_v, acc_sh.at[pl.ds(s * NPS, NPS)])
    plsc.subcore_barrier()

    pltpu.sync_copy(dst_hbm.at[wid], dst_v)
    pltpu.sync_copy(w_hbm.at[wid], w_v)

    @pl.loop(0, NCH)
    def _(ci):
        pltpu.sync_copy(w_v.at[ci], acc_sh.at[dst_v.at[ci]], add=True)

    plsc.subcore_barrier()
    pltpu.sync_copy(acc_sh.at[pl.ds(s * NPS, NPS)],
                    out_hbm.at[c, pl.ds(s * NPS, NPS)])


@functools.partial(
    pl.kernel,
    out_type=jax.ShapeDtypeStruct((NC, NP), _f32),
    mesh=_mesh,
    scratch_types=[
        pltpu.VMEM((NCH, B), jnp.int32),
        pltpu.VMEM((NCH, B), _f32),
        pltpu.VMEM((NPS,), _f32),
        pltpu.VMEM_SHARED((NP,), _f32),
    ],
    compiler_params=_sc_params,
)
def _deg_kernel(dst_hbm, w_hbm, out_hbm, dst_v, w_v, z_v, acc_sh):
    _deg_body(dst_hbm, w_hbm, out_hbm, dst_v, w_v, z_v, acc_sh)


NBUF = 5


def _msg_body(wd, hs_hbm, src_hbm, dst_hbm, w_hbm, out_hbm,
              src_v, dst_v, w_v, srows, rows, z_v, acc_sh,
              gsem, ssem, esem):
    c = lax.axis_index("c")
    s = lax.axis_index("s")
    wid = s * NC + c

    @pl.loop(0, B)
    def _(r):
        for q in range(wd // 16):
            z_v[r, pl.ds(q * 16, 16)] = jnp.zeros((16,), _f32)

    pltpu.async_copy(src_hbm.at[wid], src_v, esem.at[0])
    pltpu.async_copy(dst_hbm.at[wid], dst_v, esem.at[1])
    pltpu.async_copy(w_hbm.at[wid], w_v, esem.at[2])
    for j in range(NPS // B):
        zsem = gsem.at[j] if j < NBUF else ssem.at[j - NBUF]
        pltpu.async_copy(z_v, acc_sh.at[pl.ds(s * NPS + j * B, B)], zsem)
    for j in range(NPS // B):
        zsem = gsem.at[j] if j < NBUF else ssem.at[j - NBUF]
        pltpu.make_async_copy(z_v, acc_sh.at[pl.ds(0, B)], zsem).wait()

    plsc.subcore_barrier()

    pltpu.make_async_copy(src_hbm.at[0], src_v, esem.at[0]).wait()
    pltpu.make_async_copy(dst_hbm.at[0], dst_v, esem.at[1]).wait()
    pltpu.make_async_copy(w_hbm.at[0], w_v, esem.at[2]).wait()

    def gstart(ci, b):
        pltpu.async_copy(hs_hbm.at[src_v.at[ci]], srows.at[b], gsem.at[b])

    def gwait(b):
        pltpu.make_async_copy(hs_hbm.at[src_v.at[0]], srows.at[b],
                              gsem.at[b]).wait()

    def sstart(ci, b):
        pltpu.async_copy(rows.at[b], acc_sh.at[dst_v.at[ci]], ssem.at[b],
                         add=True)

    def swait(b):
        pltpu.make_async_copy(rows.at[b], acc_sh.at[dst_v.at[0]],
                              ssem.at[b]).wait()

    def scale(ci, b):
        for g in range(B // 16):
            wv = w_v[ci, pl.ds(g * 16, 16)]
            for j in range(16):
                splat = _lane_splat(wv, j)
                r = g * 16 + j
                for q in range(wd // 16):
                    rows[b, r, pl.ds(q * 16, 16)] = (
                        srows[b, r, pl.ds(q * 16, 16)] * splat)

    for b in range(NBUF):
        gstart(b, b)

    @pl.loop(0, NCH - NBUF, step=NBUF)
    def _(c0):
        for b in range(NBUF):
            gwait(b)
            scale(c0 + b, b)
            sstart(c0 + b, b)
        for b in range(NBUF):
            swait(b)
            gstart(c0 + NBUF + b, b)

    for b in range(NBUF):
        gwait(b)
        scale(NCH - NBUF + b, b)
        sstart(NCH - NBUF + b, b)
    for b in range(NBUF):
        swait(b)

    plsc.subcore_barrier()
    pltpu.sync_copy(acc_sh.at[pl.ds(s * NPS, NPS)],
                    out_hbm.at[c, pl.ds(s * NPS, NPS)])


def _make_msg_kernel(wd):
    @functools.partial(
        pl.kernel,
        out_type=jax.ShapeDtypeStruct((NC, NP, wd), _f32),
        mesh=_mesh,
        scratch_types=[
            pltpu.VMEM((NCH, B), jnp.int32),
            pltpu.VMEM((NCH, B), jnp.int32),
            pltpu.VMEM((NCH, B), _f32),
            pltpu.VMEM((NBUF, B, wd), _f32),
            pltpu.VMEM((NBUF, B, wd), _f32),
            pltpu.VMEM((B, wd), _f32),
            pltpu.VMEM_SHARED((NP, wd), _f32),
            pltpu.SemaphoreType.DMA((NBUF,)),
            pltpu.SemaphoreType.DMA((NBUF,)),
            pltpu.SemaphoreType.DMA((3,)),
        ],
        compiler_params=_sc_params,
    )
    def _k(hs_hbm, src_hbm, dst_hbm, w_hbm, out_hbm,
           src_v, dst_v, w_v, srows, rows, z_v, acc_sh, gsem, ssem, esem):
        _msg_body(wd, hs_hbm, src_hbm, dst_hbm, w_hbm, out_hbm,
                  src_v, dst_v, w_v, srows, rows, z_v, acc_sh,
                  gsem, ssem, esem)
    return _k


_msg_kernel_h = _make_msg_kernel(H)
_msg_kernel_c = _make_msg_kernel(CP)


def _tc0_body(x_ref, w1_ref, h_ref):
    h_ref[...] = jnp.dot(x_ref[...], w1_ref[...],
                         preferred_element_type=_f32)


def _tc1_body(h_ref, degp_ref, hs_ref):
    deg = degp_ref[0] + degp_ref[1] + 1.0
    dinv = lax.rsqrt(deg)
    hs_ref[...] = h_ref[...] * dinv


def _tc2_body(acc_ref, hs_ref, degp_ref, w2_ref, b1_ref, hs2_ref):
    deg = degp_ref[0] + degp_ref[1] + 1.0
    dinv = lax.rsqrt(deg)
    t = jax.nn.relu(dinv * (acc_ref[0] + acc_ref[1] + hs_ref[...])
                    + b1_ref[...])
    hs2_ref[...] = jnp.dot(t, w2_ref[...], preferred_element_type=_f32) * dinv


def _tc3_body(acc_ref, hs2_ref, degp_ref, b2_ref, lp_ref, xo_ref):
    deg = degp_ref[0] + degp_ref[1] + 1.0
    dinv = lax.rsqrt(deg)
    xo = dinv * (acc_ref[0] + acc_ref[1] + hs2_ref[...]) + b2_ref[...]
    col = lax.broadcasted_iota(jnp.int32, (NP, CP), 1)
    masked = jnp.where(col < C, xo, -1e30)
    m = jnp.max(masked, axis=1, keepdims=True)
    ssum = jnp.sum(jnp.exp(masked - m), axis=1, keepdims=True)
    lp_ref[...] = xo - m - jnp.log(ssum)
    xo_ref[...] = xo


def kernel(x, edge_index, e_w, idx, W1, b1, W2, b2):
    w = jnp.where(idx == 0, jnp.ones((E,), x.dtype), e_w)
    src3 = edge_index[0].reshape(NW, NCH, B)
    dst3 = edge_index[1].reshape(NW, NCH, B)
    w3 = w.reshape(NW, NCH, B)

    x_pad = jnp.pad(x, ((0, NP - N), (0, 0)))
    w2p = jnp.pad(W2, ((0, 0), (0, CP - C)))
    b1r = b1.reshape(1, H)
    b2r = jnp.pad(b2, (0, CP - C)).reshape(1, CP)

    h = pl.pallas_call(
        _tc0_body,
        out_shape=jax.ShapeDtypeStruct((NP, H), _f32),
    )(x_pad, W1)

    degp = _deg_kernel(dst3, w3)
    degp3 = degp.reshape(NC, NP, 1)

    hs = pl.pallas_call(
        _tc1_body,
        out_shape=jax.ShapeDtypeStruct((NP, H), _f32),
    )(h, degp3)

    acc1 = _msg_kernel_h(hs, src3, dst3, w3)

    hs2 = pl.pallas_call(
        _tc2_body,
        out_shape=jax.ShapeDtypeStruct((NP, CP), _f32),
    )(acc1, hs, degp3, w2p, b1r)

    acc2 = _msg_kernel_c(hs2, src3, dst3, w3)

    lp, xo = pl.pallas_call(
        _tc3_body,
        out_shape=[jax.ShapeDtypeStruct((NP, CP), _f32),
                   jax.ShapeDtypeStruct((NP, CP), _f32)],
    )(acc2, hs2, degp3, b2r)

    log_probs = lp[:N, :C]
    x_out = xo[:N, :C]
    preg = jnp.asarray(0.0, dtype=_f32)
    return (log_probs, x_out, preg)


# speedup vs baseline: 1.6396x; 1.0220x over previous
import dataclasses
import functools

import jax
import jax.numpy as jnp
from jax import lax
from jax.experimental import pallas as pl
from jax.experimental.pallas import tpu as pltpu
from jax.experimental.pallas import tpu_sc as plsc

N = 10000
NP = 10240
E = 320000
D = 128
H = 64
C = 10
CP = 16

NC = 2
NS = 16
NW = NC * NS
EPW = E // NW
B = 80
NCH = EPW // B
NPS = NP // NS

_mesh = plsc.VectorSubcoreMesh(core_axis_name="c", subcore_axis_name="s")
_f32 = jnp.float32

_sc_params = pltpu.CompilerParams(
    needs_layout_passes=False, use_tc_tiling_on_sc=False)


def _lane_splat(vec16, j):
    return lax.gather(
        vec16,
        jnp.full((16, 1), j, jnp.int32),
        lax.GatherDimensionNumbers(
            offset_dims=(), collapsed_slice_dims=(0,), start_index_map=(0,)),
        (1,),
        mode=lax.GatherScatterMode.PROMISE_IN_BOUNDS,
    )


def _deg_body(dst_hbm, w_hbm, out_hbm, dst_v, w_v, z_v, acc_sh):
    c = lax.axis_index("c")
    s = lax.axis_index("s")
    wid = s * NC + c

    @pl.loop(0, NPS, step=16)
    def _(i):
        z_v[pl.ds(i, 16)] = jnp.zeros((16,), _f32)

    pltpu.sync_copy(z_v, acc_sh.at[pl.ds(s * NPS, NPS)])
    plsc.subcore_barrier()

    pltpu.sync_copy(dst_hbm.at[wid], dst_v)
    pltpu.sync_copy(w_hbm.at[wid], w_v)

    @pl.loop(0, NCH)
    def _(ci):
        pltpu.sync_copy(w_v.at[ci], acc_sh.at[dst_v.at[ci]], add=True)

    plsc.subcore_barrier()
    pltpu.sync_copy(acc_sh.at[pl.ds(s * NPS, NPS)],
                    out_hbm.at[c, pl.ds(s * NPS, NPS)])


@functools.partial(
    pl.kernel,
    out_type=jax.ShapeDtypeStruct((NC, NP), _f32),
    mesh=_mesh,
    scratch_types=[
        pltpu.VMEM((NCH, B), jnp.int32),
        pltpu.VMEM((NCH, B), _f32),
        pltpu.VMEM((NPS,), _f32),
        pltpu.VMEM_SHARED((NP,), _f32),
    ],
    compiler_params=_sc_params,
)
def _deg_kernel(dst_hbm, w_hbm, out_hbm, dst_v, w_v, z_v, acc_sh):
    _deg_body(dst_hbm, w_hbm, out_hbm, dst_v, w_v, z_v, acc_sh)


NBUF = 5


def _msg_body(wd, hs_hbm, src_hbm, dst_hbm, w_hbm, out_hbm,
              src_v, dst_v, w_v, srows, rows, z_v, acc_sh,
              gsem, ssem, esem):
    c = lax.axis_index("c")
    s = lax.axis_index("s")
    wid = s * NC + c

    @pl.loop(0, B)
    def _(r):
        for q in range(wd // 16):
            z_v[r, pl.ds(q * 16, 16)] = jnp.zeros((16,), _f32)

    pltpu.async_copy(src_hbm.at[wid], src_v, esem.at[0])
    pltpu.async_copy(dst_hbm.at[wid], dst_v, esem.at[1])
    pltpu.async_copy(w_hbm.at[wid], w_v, esem.at[2])
    for j in range(NPS // B):
        zsem = gsem.at[j] if j < NBUF else ssem.at[j - NBUF]
        pltpu.async_copy(z_v, acc_sh.at[pl.ds(s * NPS + j * B, B)], zsem)
    for j in range(NPS // B):
        zsem = gsem.at[j] if j < NBUF else ssem.at[j - NBUF]
        pltpu.make_async_copy(z_v, acc_sh.at[pl.ds(0, B)], zsem).wait()

    plsc.subcore_barrier()

    pltpu.make_async_copy(src_hbm.at[0], src_v, esem.at[0]).wait()
    pltpu.make_async_copy(dst_hbm.at[0], dst_v, esem.at[1]).wait()
    pltpu.make_async_copy(w_hbm.at[0], w_v, esem.at[2]).wait()

    def gstart(ci, b):
        pltpu.async_copy(hs_hbm.at[src_v.at[ci]], srows.at[b], gsem.at[b])

    def gwait(b):
        pltpu.make_async_copy(hs_hbm.at[src_v.at[0]], srows.at[b],
                              gsem.at[b]).wait()

    def sstart(ci, b):
        pltpu.async_copy(rows.at[b], acc_sh.at[dst_v.at[ci]], ssem.at[b],
                         add=True)

    def swait(b):
        pltpu.make_async_copy(rows.at[b], acc_sh.at[dst_v.at[0]],
                              ssem.at[b]).wait()

    def scale(ci, b):
        for g in range(B // 16):
            wv = w_v[ci, pl.ds(g * 16, 16)]
            for j in range(16):
                splat = _lane_splat(wv, j)
                r = g * 16 + j
                for q in range(wd // 16):
                    rows[b, r, pl.ds(q * 16, 16)] = (
                        srows[b, r, pl.ds(q * 16, 16)] * splat)

    for b in range(NBUF):
        gstart(b, b)

    @pl.loop(0, NCH - NBUF, step=NBUF)
    def _(c0):
        for b in range(NBUF):
            gwait(b)
            scale(c0 + b, b)
            sstart(c0 + b, b)
        for b in range(NBUF):
            swait(b)
            gstart(c0 + NBUF + b, b)

    for b in range(NBUF):
        gwait(b)
        scale(NCH - NBUF + b, b)
        sstart(NCH - NBUF + b, b)
    for b in range(NBUF):
        swait(b)

    plsc.subcore_barrier()
    pltpu.sync_copy(acc_sh.at[pl.ds(s * NPS, NPS)],
                    out_hbm.at[c, pl.ds(s * NPS, NPS)])


def _make_msg_kernel(wd):
    @functools.partial(
        pl.kernel,
        out_type=jax.ShapeDtypeStruct((NC, NP, wd), _f32),
        mesh=_mesh,
        scratch_types=[
            pltpu.VMEM((NCH, B), jnp.int32),
            pltpu.VMEM((NCH, B), jnp.int32),
            pltpu.VMEM((NCH, B), _f32),
            pltpu.VMEM((NBUF, B, wd), _f32),
            pltpu.VMEM((NBUF, B, wd), _f32),
            pltpu.VMEM((B, wd), _f32),
            pltpu.VMEM_SHARED((NP, wd), _f32),
            pltpu.SemaphoreType.DMA((NBUF,)),
            pltpu.SemaphoreType.DMA((NBUF,)),
            pltpu.SemaphoreType.DMA((3,)),
        ],
        compiler_params=_sc_params,
    )
    def _k(hs_hbm, src_hbm, dst_hbm, w_hbm, out_hbm,
           src_v, dst_v, w_v, srows, rows, z_v, acc_sh, gsem, ssem, esem):
        _msg_body(wd, hs_hbm, src_hbm, dst_hbm, w_hbm, out_hbm,
                  src_v, dst_v, w_v, srows, rows, z_v, acc_sh,
                  gsem, ssem, esem)
    return _k


_msg_kernel_h = _make_msg_kernel(H)
_msg_kernel_c = _make_msg_kernel(CP)


def _tc0_body(x_ref, w1_ref, h_ref):
    h_ref[...] = jnp.dot(x_ref[...], w1_ref[...],
                         preferred_element_type=_f32)


def _tc1_body(h_ref, degp_ref, hs_ref):
    deg = degp_ref[0, 0:N] + degp_ref[1, 0:N] + 1.0
    dinv = lax.rsqrt(deg)
    hs_ref[...] = h_ref[...] * dinv


def _tc2_body(acc_ref, hs_ref, degp_ref, w2_ref, b1_ref, hs2_ref):
    deg = degp_ref[0, 0:N] + degp_ref[1, 0:N] + 1.0
    dinv = lax.rsqrt(deg)
    t = jax.nn.relu(dinv * (acc_ref[0, 0:N] + acc_ref[1, 0:N] + hs_ref[...])
                    + b1_ref[...])
    hs2_ref[...] = jnp.dot(t, w2_ref[...], preferred_element_type=_f32) * dinv


def _tc3_body(acc_ref, hs2_ref, degp_ref, b2_ref, lp_ref, xo_ref):
    deg = degp_ref[0, 0:N] + degp_ref[1, 0:N] + 1.0
    dinv = lax.rsqrt(deg)
    xo = dinv * (acc_ref[0, 0:N] + acc_ref[1, 0:N] + hs2_ref[...]) \
        + b2_ref[...]
    col = lax.broadcasted_iota(jnp.int32, (N, CP), 1)
    masked = jnp.where(col < C, xo, -1e30)
    m = jnp.max(masked, axis=1, keepdims=True)
    ssum = jnp.sum(jnp.exp(masked - m), axis=1, keepdims=True)
    lp_ref[...] = (xo - m - jnp.log(ssum))[:, 0:C]
    xo_ref[...] = xo[:, 0:C]


def kernel(x, edge_index, e_w, idx, W1, b1, W2, b2):
    w = jnp.where(idx == 0, jnp.ones((E,), x.dtype), e_w)
    src3 = edge_index[0].reshape(NW, NCH, B)
    dst3 = edge_index[1].reshape(NW, NCH, B)
    w3 = w.reshape(NW, NCH, B)

    w2p = jnp.pad(W2, ((0, 0), (0, CP - C)))
    b1r = b1.reshape(1, H)
    b2r = jnp.pad(b2, (0, CP - C)).reshape(1, CP)

    h = pl.pallas_call(
        _tc0_body,
        out_shape=jax.ShapeDtypeStruct((N, H), _f32),
    )(x, W1)

    degp = _deg_kernel(dst3, w3)
    degp3 = degp.reshape(NC, NP, 1)

    hs = pl.pallas_call(
        _tc1_body,
        out_shape=jax.ShapeDtypeStruct((N, H), _f32),
    )(h, degp3)

    acc1 = _msg_kernel_h(hs, src3, dst3, w3)

    hs2 = pl.pallas_call(
        _tc2_body,
        out_shape=jax.ShapeDtypeStruct((N, CP), _f32),
    )(acc1, hs, degp3, w2p, b1r)

    acc2 = _msg_kernel_c(hs2, src3, dst3, w3)

    log_probs, x_out = pl.pallas_call(
        _tc3_body,
        out_shape=[jax.ShapeDtypeStruct((N, C), _f32),
                   jax.ShapeDtypeStruct((N, C), _f32)],
    )(acc2, hs2, degp3, b2r)

    preg = jnp.asarray(0.0, dtype=_f32)
    return (log_probs, x_out, preg)


# speedup vs baseline: 1.6397x; 1.0001x over previous
import functools

import jax
import jax.numpy as jnp
from jax import lax
from jax.experimental import pallas as pl
from jax.experimental.pallas import tpu as pltpu
from jax.experimental.pallas import tpu_sc as plsc

N = 10000
NP = 10240
E = 320000
D = 128
H = 64
C = 10
CP = 16

NC = 2
NS = 16
NW = NC * NS
EPW = E // NW
B = 80
NCH = EPW // B
NPS = NP // NS

_mesh = plsc.VectorSubcoreMesh(core_axis_name="c", subcore_axis_name="s")
_f32 = jnp.float32

_sc_params = pltpu.CompilerParams(
    needs_layout_passes=False, use_tc_tiling_on_sc=False)


def _lane_splat(vec16, j):
    return lax.gather(
        vec16,
        jnp.full((16, 1), j, jnp.int32),
        lax.GatherDimensionNumbers(
            offset_dims=(), collapsed_slice_dims=(0,), start_index_map=(0,)),
        (1,),
        mode=lax.GatherScatterMode.PROMISE_IN_BOUNDS,
    )


def _deg_body(dst_hbm, w_hbm, out_hbm, dst_v, w_v, z_v, acc_sh):
    c = lax.axis_index("c")
    s = lax.axis_index("s")
    wid = s * NC + c

    @pl.loop(0, NPS, step=16)
    def _(i):
        z_v[pl.ds(i, 16)] = jnp.zeros((16,), _f32)

    pltpu.sync_copy(z_v, acc_sh.at[pl.ds(s * NPS, NPS)])
    plsc.subcore_barrier()

    pltpu.sync_copy(dst_hbm.at[wid], dst_v)
    pltpu.sync_copy(w_hbm.at[wid], w_v)

    @pl.loop(0, NCH)
    def _(ci):
        pltpu.sync_copy(w_v.at[ci], acc_sh.at[dst_v.at[ci]], add=True)

    plsc.subcore_barrier()
    pltpu.sync_copy(acc_sh.at[pl.ds(s * NPS, NPS)],
                    out_hbm.at[c, pl.ds(s * NPS, NPS)])


@functools.partial(
    pl.kernel,
    out_type=jax.ShapeDtypeStruct((NC, NP), _f32),
    mesh=_mesh,
    scratch_types=[
        pltpu.VMEM((NCH, B), jnp.int32),
        pltpu.VMEM((NCH, B), _f32),
        pltpu.VMEM((NPS,), _f32),
        pltpu.VMEM_SHARED((NP,), _f32),
    ],
    compiler_params=_sc_params,
)
def _deg_kernel(dst_hbm, w_hbm, out_hbm, dst_v, w_v, z_v, acc_sh):
    _deg_body(dst_hbm, w_hbm, out_hbm, dst_v, w_v, z_v, acc_sh)


NBUF = 5


def _msg_body(wd, hs_hbm, src_hbm, dst_hbm, w_hbm, out_hbm,
              src_v, dst_v, w_v, srows, rows, z_v, acc_sh,
              gsem, ssem, esem):
    c = lax.axis_index("c")
    s = lax.axis_index("s")
    wid = s * NC + c

    @pl.loop(0, B)
    def _(r):
        for q in range(wd // 16):
            z_v[r, pl.ds(q * 16, 16)] = jnp.zeros((16,), _f32)

    pltpu.async_copy(src_hbm.at[wid], src_v, esem.at[0])
    pltpu.async_copy(dst_hbm.at[wid], dst_v, esem.at[1])
    pltpu.async_copy(w_hbm.at[wid], w_v, esem.at[2])
    for j in range(NPS // B):
        zsem = gsem.at[j] if j < NBUF else ssem.at[j - NBUF]
        pltpu.async_copy(z_v, acc_sh.at[pl.ds(s * NPS + j * B, B)], zsem)
    for j in range(NPS // B):
        zsem = gsem.at[j] if j < NBUF else ssem.at[j - NBUF]
        pltpu.make_async_copy(z_v, acc_sh.at[pl.ds(0, B)], zsem).wait()

    plsc.subcore_barrier()

    pltpu.make_async_copy(src_hbm.at[0], src_v, esem.at[0]).wait()
    pltpu.make_async_copy(dst_hbm.at[0], dst_v, esem.at[1]).wait()
    pltpu.make_async_copy(w_hbm.at[0], w_v, esem.at[2]).wait()

    def gstart(ci, b):
        pltpu.async_copy(hs_hbm.at[src_v.at[ci]], srows.at[b], gsem.at[b])

    def gwait(b):
        pltpu.make_async_copy(hs_hbm.at[src_v.at[0]], srows.at[b],
                              gsem.at[b]).wait()

    def sstart(ci, b):
        pltpu.async_copy(rows.at[b], acc_sh.at[dst_v.at[ci]], ssem.at[b],
                         add=True)

    def swait(b):
        pltpu.make_async_copy(rows.at[b], acc_sh.at[dst_v.at[0]],
                              ssem.at[b]).wait()

    def scale(ci, b):
        for g in range(B // 16):
            wv = w_v[ci, pl.ds(g * 16, 16)]
            for j in range(16):
                splat = _lane_splat(wv, j)
                r = g * 16 + j
                for q in range(wd // 16):
                    rows[b, r, pl.ds(q * 16, 16)] = (
                        srows[b, r, pl.ds(q * 16, 16)] * splat)

    for b in range(NBUF):
        gstart(b, b)

    @pl.loop(0, NCH - NBUF, step=NBUF)
    def _(c0):
        for b in range(NBUF):
            gwait(b)
            scale(c0 + b, b)
            sstart(c0 + b, b)
        for b in range(NBUF):
            swait(b)
            gstart(c0 + NBUF + b, b)

    for b in range(NBUF):
        gwait(b)
        scale(NCH - NBUF + b, b)
        sstart(NCH - NBUF + b, b)
    for b in range(NBUF):
        swait(b)

    plsc.subcore_barrier()
    pltpu.sync_copy(acc_sh.at[pl.ds(s * NPS, NPS)],
                    out_hbm.at[c, pl.ds(s * NPS, NPS)])


def _make_msg_kernel(wd):
    @functools.partial(
        pl.kernel,
        out_type=jax.ShapeDtypeStruct((NC, NP, wd), _f32),
        mesh=_mesh,
        scratch_types=[
            pltpu.VMEM((NCH, B), jnp.int32),
            pltpu.VMEM((NCH, B), jnp.int32),
            pltpu.VMEM((NCH, B), _f32),
            pltpu.VMEM((NBUF, B, wd), _f32),
            pltpu.VMEM((NBUF, B, wd), _f32),
            pltpu.VMEM((B, wd), _f32),
            pltpu.VMEM_SHARED((NP, wd), _f32),
            pltpu.SemaphoreType.DMA((NBUF,)),
            pltpu.SemaphoreType.DMA((NBUF,)),
            pltpu.SemaphoreType.DMA((3,)),
        ],
        compiler_params=_sc_params,
    )
    def _k(hs_hbm, src_hbm, dst_hbm, w_hbm, out_hbm,
           src_v, dst_v, w_v, srows, rows, z_v, acc_sh, gsem, ssem, esem):
        _msg_body(wd, hs_hbm, src_hbm, dst_hbm, w_hbm, out_hbm,
                  src_v, dst_v, w_v, srows, rows, z_v, acc_sh,
                  gsem, ssem, esem)
    return _k


_msg_kernel_h = _make_msg_kernel(H)
_msg_kernel_c = _make_msg_kernel(CP)


def _tc0_body(x_ref, w1_ref, h_ref):
    h_ref[...] = jnp.dot(x_ref[...], w1_ref[...],
                         preferred_element_type=_f32)


def _tc1_body(h_ref, degp_ref, hs_ref):
    deg = degp_ref[0, 0:N] + degp_ref[1, 0:N] + 1.0
    dinv = lax.rsqrt(deg)
    hs_ref[...] = h_ref[...] * dinv


def _tc2_body(acc_ref, hs_ref, degp_ref, w2_ref, b1_ref, hs2_ref):
    deg = degp_ref[0, 0:N] + degp_ref[1, 0:N] + 1.0
    dinv = lax.rsqrt(deg)
    t = jax.nn.relu(dinv * (acc_ref[0, 0:N] + acc_ref[1, 0:N] + hs_ref[...])
                    + b1_ref[...])
    hs2_ref[...] = jnp.dot(t, w2_ref[...], preferred_element_type=_f32) * dinv


def _tc3_body(acc_ref, hs2_ref, degp_ref, b2_ref, lp_ref, xo_ref):
    deg = degp_ref[0, 0:N] + degp_ref[1, 0:N] + 1.0
    dinv = lax.rsqrt(deg)
    xo = dinv * (acc_ref[0, 0:N] + acc_ref[1, 0:N] + hs2_ref[...]) \
        + b2_ref[...]
    col = lax.broadcasted_iota(jnp.int32, (N, CP), 1)
    masked = jnp.where(col < C, xo, -1e30)
    m = jnp.max(masked, axis=1, keepdims=True)
    ssum = jnp.sum(jnp.exp(masked - m), axis=1, keepdims=True)
    lp_ref[...] = (xo - m - jnp.log(ssum))[:, 0:C]
    xo_ref[...] = xo[:, 0:C]


def kernel(x, edge_index, e_w, idx, W1, b1, W2, b2):
    w = jnp.where(idx == 0, jnp.ones((E,), x.dtype), e_w)
    src3 = edge_index[0].reshape(NW, NCH, B)
    dst3 = edge_index[1].reshape(NW, NCH, B)
    w3 = w.reshape(NW, NCH, B)

    w2p = jnp.pad(W2, ((0, 0), (0, CP - C)))
    b1r = b1.reshape(1, H)
    b2r = jnp.pad(b2, (0, CP - C)).reshape(1, CP)

    h = pl.pallas_call(
        _tc0_body,
        out_shape=jax.ShapeDtypeStruct((N, H), _f32),
    )(x, W1)

    degp = _deg_kernel(dst3, w3)
    degp3 = degp.reshape(NC, NP, 1)

    hs = pl.pallas_call(
        _tc1_body,
        out_shape=jax.ShapeDtypeStruct((N, H), _f32),
    )(h, degp3)

    acc1 = _msg_kernel_h(hs, src3, dst3, w3)

    hs2 = pl.pallas_call(
        _tc2_body,
        out_shape=jax.ShapeDtypeStruct((N, CP), _f32),
    )(acc1, hs, degp3, w2p, b1r)

    acc2 = _msg_kernel_c(hs2, src3, dst3, w3)

    log_probs, x_out = pl.pallas_call(
        _tc3_body,
        out_shape=[jax.ShapeDtypeStruct((N, C), _f32),
                   jax.ShapeDtypeStruct((N, C), _f32)],
    )(acc2, hs2, degp3, b2r)

    preg = jnp.asarray(0.0, dtype=_f32)
    return (log_probs, x_out, preg)
